# Initial kernel scaffold; baseline (speedup 1.0000x reference)
import functools

import jax
import jax.numpy as jnp
from jax import lax
from jax.experimental import pallas as pl
from jax.experimental.pallas import tpu as pltpu
from jax.experimental.pallas import tpu_sc as plsc

N = 100000
T = 2
NUM_STATIC = 3
NUM_DYN = 2
HIDDEN = 128

NC = 2
NS = 16
NW = NC * NS

G = 128
K = 8
SUPER = K * G
NITER = 98
EDGES_PER_W = SUPER * NITER
E_PAD = EDGES_PER_W * NW
ROWS_PER_W = EDGES_PER_W // G

N_PAD = 100096
SL = N_PAD // NS


def _sc_mesh():
    return plsc.VectorSubcoreMesh(core_axis_name="c", subcore_axis_name="s",
                                  num_cores=NC, num_subcores=NS)


@functools.partial(
    pl.kernel,
    out_type=jax.ShapeDtypeStruct((NC, N_PAD, NUM_DYN), jnp.float32),
    mesh=_sc_mesh(),
    scratch_types=[
        pltpu.VMEM((K, G), jnp.int32),
        pltpu.VMEM((K, G), jnp.int32),
        pltpu.VMEM((SUPER, NUM_DYN), jnp.float32),
        pltpu.MemorySpace.VMEM_SHARED((N_PAD, NUM_DYN), jnp.float32),
        pltpu.SemaphoreType.DMA,
    ],
)
def _sc_agg(y_hbm, srcm, dstm, zeros_hbm, parts, src_v, dst_v, rows_v, agg_sh, sem):
    c = lax.axis_index("c")
    s = lax.axis_index("s")
    wid = s * NC + c
    base_row = wid * ROWS_PER_W

    pltpu.sync_copy(zeros_hbm, agg_sh.at[pl.ds(s * SL, SL)])
    plsc.subcore_barrier()

    def body(i, carry):
        r0 = base_row + i * K
        pltpu.sync_copy(srcm.at[pl.ds(r0, K)], src_v)
        pltpu.sync_copy(dstm.at[pl.ds(r0, K)], dst_v)
        descs = [
            pltpu.async_copy(y_hbm.at[src_v.at[t]],
                             rows_v.at[pl.ds(t * G, G)], sem)
            for t in range(K)
        ]
        for d in descs:
            d.wait()
        for t in range(K):
            pltpu.sync_copy(rows_v.at[pl.ds(t * G, G)],
                            agg_sh.at[dst_v.at[t]], add=True)
        return carry

    lax.fori_loop(0, NITER, body, 0)
    plsc.subcore_barrier()
    pltpu.sync_copy(agg_sh.at[pl.ds(s * SL, SL)], parts.at[c, pl.ds(s * SL, SL)])


@functools.partial(
    pl.kernel,
    out_type=jax.ShapeDtypeStruct((NC, N_PAD, NUM_DYN), jnp.float32),
    mesh=_sc_mesh(),
    scratch_types=[
        pltpu.VMEM((K, G), jnp.int32),
        pltpu.VMEM((G, NUM_DYN), jnp.float32),
        pltpu.MemorySpace.VMEM_SHARED((N_PAD, NUM_DYN), jnp.float32),
        pltpu.SemaphoreType.DMA,
    ],
)
def _sc_deg(dstm, zeros_hbm, ones_hbm, parts, dst_v, ones_v, agg_sh, sem):
    c = lax.axis_index("c")
    s = lax.axis_index("s")
    wid = s * NC + c
    base_row = wid * ROWS_PER_W

    pltpu.sync_copy(zeros_hbm, agg_sh.at[pl.ds(s * SL, SL)])
    pltpu.sync_copy(ones_hbm, ones_v)
    plsc.subcore_barrier()

    def body(i, carry):
        r0 = base_row + i * K
        pltpu.sync_copy(dstm.at[pl.ds(r0, K)], dst_v)
        for t in range(K):
            pltpu.sync_copy(ones_v, agg_sh.at[dst_v.at[t]], add=True)
        return carry

    lax.fori_loop(0, NITER, body, 0)
    plsc.subcore_barrier()
    pltpu.sync_copy(agg_sh.at[pl.ds(s * SL, SL)], parts.at[c, pl.ds(s * SL, SL)])


R = 1000


def _mlp(s, y, p0, p1, d0, d1, w1a, w1b, w1c, b1, w2, b2):
    deg = jnp.maximum(d0 + d1, 1.0)
    a = (p0 + p1) / deg
    z = (jnp.dot(s, w1a, preferred_element_type=jnp.float32)
         + jnp.dot(y, w1b, preferred_element_type=jnp.float32)
         + jnp.dot(a, w1c, preferred_element_type=jnp.float32) + b1)
    z = jnp.maximum(z, 0.0)
    return jnp.dot(z, w2, preferred_element_type=jnp.float32) + b2


def _tc_stage_body(cy, cw, s_ref, y_ref, p0_ref, p1_ref, d0_ref, d1_ref,
                   y0_ref, acc_ref, w1a_ref, w1b_ref, w1c_ref, b1_ref,
                   w2_ref, b2_ref, ynext_ref, accout_ref):
    f = _mlp(s_ref[...], y_ref[...], p0_ref[...], p1_ref[...],
             d0_ref[...], d1_ref[...], w1a_ref[...], w1b_ref[...],
             w1c_ref[...], b1_ref[...], w2_ref[...], b2_ref[...])
    ynext_ref[...] = y0_ref[...] + cy * f
    accout_ref[...] = acc_ref[...] + cw * f


def _tc_final_body(s_ref, y_ref, p0_ref, p1_ref, d0_ref, d1_ref,
                   y0_ref, acc_ref, pos_ref, w1a_ref, w1b_ref, w1c_ref,
                   b1_ref, w2_ref, b2_ref, out_ref):
    f = _mlp(s_ref[...], y_ref[...], p0_ref[...], p1_ref[...],
             d0_ref[...], d1_ref[...], w1a_ref[...], w1b_ref[...],
             w1c_ref[...], b1_ref[...], w2_ref[...], b2_ref[...])
    pos = pos_ref[...]
    m = 0.02
    px = pos[:, 0:1]
    py = pos[:, 1:2]
    inside = (px > m) & (px < 1.0 - m) & (py > m) & (py < 1.0 - m)
    mask = jnp.where(inside, 1.0, 0.0)
    upd = mask * ((1.0 / 6.0) * (acc_ref[...] + f))
    out_ref[...] = jnp.clip(y0_ref[...] + upd, -10.0, 10.0)


def _row_spec(width):
    return pl.BlockSpec((R, width), lambda i: (i, 0))


def _full_spec(shape):
    return pl.BlockSpec(shape, lambda i: tuple(0 for _ in shape))


_W_SPECS = [
    _full_spec((NUM_STATIC, HIDDEN)),
    _full_spec((NUM_DYN, HIDDEN)),
    _full_spec((NUM_DYN, HIDDEN)),
    _full_spec((1, HIDDEN)),
    _full_spec((HIDDEN, NUM_DYN)),
    _full_spec((1, NUM_DYN)),
]

_STAGE_IN_SPECS = (
    [_row_spec(NUM_STATIC)] + [_row_spec(NUM_DYN)] * 5 + [_row_spec(NUM_DYN)] * 2
    + _W_SPECS
)


def _tc_stage(cy, cw, s, y, p0, p1, d0, d1, y0, acc, weights):
    body = functools.partial(_tc_stage_body, cy, cw)
    out_shape = [jax.ShapeDtypeStruct((N, NUM_DYN), jnp.float32)] * 2
    return pl.pallas_call(
        body,
        grid=(N // R,),
        in_specs=_STAGE_IN_SPECS,
        out_specs=[_row_spec(NUM_DYN)] * 2,
        out_shape=out_shape,
    )(s, y, p0, p1, d0, d1, y0, acc, *weights)


def _tc_final(s, y, p0, p1, d0, d1, y0, acc, pos, weights):
    in_specs = (
        [_row_spec(NUM_STATIC)] + [_row_spec(NUM_DYN)] * 5
        + [_row_spec(NUM_DYN)] * 2 + [_row_spec(2)] + _W_SPECS
    )
    return pl.pallas_call(
        _tc_final_body,
        grid=(N // R,),
        in_specs=in_specs,
        out_specs=_row_spec(NUM_DYN),
        out_shape=jax.ShapeDtypeStruct((N, NUM_DYN), jnp.float32),
    )(s, y, p0, p1, d0, d1, y0, acc, pos, *weights)


def kernel(x, edge_index, W1, b1, W2, b2):
    ei = edge_index.astype(jnp.int32)
    pad = E_PAD - ei.shape[1]
    src = jnp.concatenate([ei[0], jnp.zeros((pad,), jnp.int32)])
    dst = jnp.concatenate([ei[1], jnp.full((pad,), N, jnp.int32)])
    srcm = src.reshape(E_PAD // G, G)
    dstm = dst.reshape(E_PAD // G, G)

    zeros_sl = jnp.zeros((SL, NUM_DYN), jnp.float32)
    ones_g = jnp.ones((G, NUM_DYN), jnp.float32)

    dparts = _sc_deg(dstm, zeros_sl, ones_g)
    d0 = dparts[0, :N]
    d1 = dparts[1, :N]

    weights = (W1[:NUM_STATIC], W1[NUM_STATIC:NUM_STATIC + NUM_DYN],
               W1[NUM_STATIC + NUM_DYN:], b1.reshape(1, HIDDEN),
               W2, b2.reshape(1, NUM_DYN))

    pos = x[0][:, :2]
    acc0 = jnp.zeros((N, NUM_DYN), jnp.float32)

    preds = []
    y_prev = None
    for t in range(T):
        xt = x[t]
        static = xt[:, :NUM_STATIC]
        dyn = xt[:, NUM_STATIC:] if y_prev is None else y_prev
        y0 = jnp.clip(dyn, -10.0, 10.0)

        p = _sc_agg(y0, srcm, dstm, zeros_sl)
        ya, acc = _tc_stage(0.5, 1.0, static, y0, p[0, :N], p[1, :N],
                            d0, d1, y0, acc0, weights)
        p = _sc_agg(ya, srcm, dstm, zeros_sl)
        yb, acc = _tc_stage(0.5, 2.0, static, ya, p[0, :N], p[1, :N],
                            d0, d1, y0, acc, weights)
        p = _sc_agg(yb, srcm, dstm, zeros_sl)
        yc, acc = _tc_stage(1.0, 2.0, static, yb, p[0, :N], p[1, :N],
                            d0, d1, y0, acc, weights)
        p = _sc_agg(yc, srcm, dstm, zeros_sl)
        f_next = _tc_final(static, yc, p[0, :N], p[1, :N],
                           d0, d1, y0, acc, pos, weights)
        preds.append(f_next)
        y_prev = f_next

    return jnp.stack(preds, axis=0)


# speedup vs baseline: 24.6533x; 24.6533x over previous
import functools

import jax
import jax.numpy as jnp
from jax import lax
from jax.experimental import pallas as pl
from jax.experimental.pallas import tpu as pltpu
from jax.experimental.pallas import tpu_sc as plsc

N = 100000
T = 2
NUM_STATIC = 3
NUM_DYN = 2
HIDDEN = 128
DW = 8

NC = 2
NS = 16
NW = NC * NS

G = 128
K = 8
SUPER = K * G
NITER = 98
EDGES_PER_W = SUPER * NITER
E_PAD = EDGES_PER_W * NW
ROWS_PER_W = EDGES_PER_W // G

N_PAD = 100096
SL = N_PAD // NS


def _sc_mesh():
    return plsc.VectorSubcoreMesh(core_axis_name="c", subcore_axis_name="s",
                                  num_cores=NC, num_subcores=NS)


@functools.cache
def _make_sc_agg():
    return functools.partial(
        pl.kernel,
        out_type=jax.ShapeDtypeStruct((NC, N_PAD, DW), jnp.float32),
        mesh=_sc_mesh(),
        scratch_types=[
            pltpu.VMEM((K, G), jnp.int32),
            pltpu.VMEM((K, G), jnp.int32),
            pltpu.VMEM((SUPER, DW), jnp.float32),
            pltpu.MemorySpace.VMEM_SHARED((N_PAD, DW), jnp.float32),
            pltpu.SemaphoreType.DMA,
        ],
        compiler_params=pltpu.CompilerParams(use_tc_tiling_on_sc=False),
    )(_sc_agg_body)


def _sc_agg(y8, srcm, dstm, zeros_sl):
    return _make_sc_agg()(y8, srcm, dstm, zeros_sl)


def _sc_agg_body(y_hbm, srcm, dstm, zeros_hbm, parts, src_v, dst_v, rows_v,
                 agg_sh, sem):
    c = lax.axis_index("c")
    s = lax.axis_index("s")
    wid = s * NC + c
    base_row = wid * ROWS_PER_W

    pltpu.sync_copy(zeros_hbm, agg_sh.at[pl.ds(s * SL, SL)])
    plsc.subcore_barrier()

    def body(i, carry):
        r0 = base_row + i * K
        pltpu.sync_copy(srcm.at[pl.ds(r0, K)], src_v)
        pltpu.sync_copy(dstm.at[pl.ds(r0, K)], dst_v)
        descs = [
            pltpu.async_copy(y_hbm.at[src_v.at[t]],
                             rows_v.at[pl.ds(t * G, G)], sem)
            for t in range(K)
        ]
        for d in descs:
            d.wait()
        for t in range(K):
            pltpu.sync_copy(rows_v.at[pl.ds(t * G, G)],
                            agg_sh.at[dst_v.at[t]], add=True)
        return carry

    lax.fori_loop(0, NITER, body, 0)
    plsc.subcore_barrier()
    pltpu.sync_copy(agg_sh.at[pl.ds(s * SL, SL)], parts.at[c, pl.ds(s * SL, SL)])


@functools.cache
def _make_sc_deg():
    return functools.partial(
        pl.kernel,
        out_type=jax.ShapeDtypeStruct((NC, N_PAD, DW), jnp.float32),
        mesh=_sc_mesh(),
        scratch_types=[
            pltpu.VMEM((K, G), jnp.int32),
            pltpu.VMEM((G, DW), jnp.float32),
            pltpu.MemorySpace.VMEM_SHARED((N_PAD, DW), jnp.float32),
            pltpu.SemaphoreType.DMA,
        ],
        compiler_params=pltpu.CompilerParams(use_tc_tiling_on_sc=False),
    )(_sc_deg_body)


def _sc_deg(dstm, zeros_sl, ones_g):
    return _make_sc_deg()(dstm, zeros_sl, ones_g)


def _sc_deg_body(dstm, zeros_hbm, ones_hbm, parts, dst_v, ones_v, agg_sh, sem):
    c = lax.axis_index("c")
    s = lax.axis_index("s")
    wid = s * NC + c
    base_row = wid * ROWS_PER_W

    pltpu.sync_copy(zeros_hbm, agg_sh.at[pl.ds(s * SL, SL)])
    pltpu.sync_copy(ones_hbm, ones_v)
    plsc.subcore_barrier()

    def body(i, carry):
        r0 = base_row + i * K
        pltpu.sync_copy(dstm.at[pl.ds(r0, K)], dst_v)
        for t in range(K):
            pltpu.sync_copy(ones_v, agg_sh.at[dst_v.at[t]], add=True)
        return carry

    lax.fori_loop(0, NITER, body, 0)
    plsc.subcore_barrier()
    pltpu.sync_copy(agg_sh.at[pl.ds(s * SL, SL)], parts.at[c, pl.ds(s * SL, SL)])


R = 1000


def _mlp(s, y8, p0, p1, d0, d1, w1a, w1b, w1c, b1, w2, b2):
    y = y8[:, :NUM_DYN]
    deg = jnp.maximum(d0[:, :NUM_DYN] + d1[:, :NUM_DYN], 1.0)
    a = (p0[:, :NUM_DYN] + p1[:, :NUM_DYN]) / deg
    z = (jnp.dot(s, w1a, preferred_element_type=jnp.float32)
         + jnp.dot(y, w1b, preferred_element_type=jnp.float32)
         + jnp.dot(a, w1c, preferred_element_type=jnp.float32) + b1)
    z = jnp.maximum(z, 0.0)
    return jnp.dot(z, w2, preferred_element_type=jnp.float32) + b2


def _pad8(v):
    return jnp.concatenate([v, jnp.zeros((v.shape[0], DW - NUM_DYN), v.dtype)],
                           axis=1)


def _tc_stage_body(cy, cw, s_ref, y_ref, p0_ref, p1_ref, d0_ref, d1_ref,
                   y0_ref, acc_ref, w1a_ref, w1b_ref, w1c_ref, b1_ref,
                   w2_ref, b2_ref, ynext_ref, accout_ref):
    f = _mlp(s_ref[...], y_ref[...], p0_ref[...], p1_ref[...],
             d0_ref[...], d1_ref[...], w1a_ref[...], w1b_ref[...],
             w1c_ref[...], b1_ref[...], w2_ref[...], b2_ref[...])
    ynext_ref[...] = _pad8(y0_ref[...][:, :NUM_DYN] + cy * f)
    accout_ref[...] = acc_ref[...] + cw * f


def _tc_final_body(s_ref, y_ref, p0_ref, p1_ref, d0_ref, d1_ref,
                   y0_ref, acc_ref, pos_ref, w1a_ref, w1b_ref, w1c_ref,
                   b1_ref, w2_ref, b2_ref, out_ref):
    f = _mlp(s_ref[...], y_ref[...], p0_ref[...], p1_ref[...],
             d0_ref[...], d1_ref[...], w1a_ref[...], w1b_ref[...],
             w1c_ref[...], b1_ref[...], w2_ref[...], b2_ref[...])
    pos = pos_ref[...]
    m = 0.02
    px = pos[:, 0:1]
    py = pos[:, 1:2]
    inside = (px > m) & (px < 1.0 - m) & (py > m) & (py < 1.0 - m)
    mask = jnp.where(inside, 1.0, 0.0)
    upd = mask * ((1.0 / 6.0) * (acc_ref[...] + f))
    out_ref[...] = _pad8(jnp.clip(y0_ref[...][:, :NUM_DYN] + upd, -10.0, 10.0))


def _row_spec(width):
    return pl.BlockSpec((R, width), lambda i: (i, 0))


def _full_spec(shape):
    return pl.BlockSpec(shape, lambda i: tuple(0 for _ in shape))


_W_SPECS = [
    _full_spec((NUM_STATIC, HIDDEN)),
    _full_spec((NUM_DYN, HIDDEN)),
    _full_spec((NUM_DYN, HIDDEN)),
    _full_spec((1, HIDDEN)),
    _full_spec((HIDDEN, NUM_DYN)),
    _full_spec((1, NUM_DYN)),
]

_STAGE_IN_SPECS = (
    [_row_spec(NUM_STATIC)] + [_row_spec(DW)] * 5 + [_row_spec(DW)]
    + [_row_spec(NUM_DYN)] + _W_SPECS
)


def _tc_stage(cy, cw, s, y8, p0, p1, d0, d1, y08, acc, weights):
    body = functools.partial(_tc_stage_body, cy, cw)
    out_shape = [jax.ShapeDtypeStruct((N, DW), jnp.float32),
                 jax.ShapeDtypeStruct((N, NUM_DYN), jnp.float32)]
    return pl.pallas_call(
        body,
        grid=(N // R,),
        in_specs=_STAGE_IN_SPECS,
        out_specs=[_row_spec(DW), _row_spec(NUM_DYN)],
        out_shape=out_shape,
    )(s, y8, p0, p1, d0, d1, y08, acc, *weights)


def _tc_final(s, y8, p0, p1, d0, d1, y08, acc, pos, weights):
    in_specs = (
        [_row_spec(NUM_STATIC)] + [_row_spec(DW)] * 5 + [_row_spec(DW)]
        + [_row_spec(NUM_DYN)] + [_row_spec(2)] + _W_SPECS
    )
    return pl.pallas_call(
        _tc_final_body,
        grid=(N // R,),
        in_specs=in_specs,
        out_specs=_row_spec(DW),
        out_shape=jax.ShapeDtypeStruct((N, DW), jnp.float32),
    )(s, y8, p0, p1, d0, d1, y08, acc, pos, *weights)


def kernel(x, edge_index, W1, b1, W2, b2):
    ei = edge_index.astype(jnp.int32)
    pad = E_PAD - ei.shape[1]
    src = jnp.concatenate([ei[0], jnp.zeros((pad,), jnp.int32)])
    dst = jnp.concatenate([ei[1], jnp.full((pad,), N, jnp.int32)])
    srcm = src.reshape(E_PAD // G, G)
    dstm = dst.reshape(E_PAD // G, G)

    zeros_sl = jnp.zeros((SL, DW), jnp.float32)
    ones_g = jnp.ones((G, DW), jnp.float32)

    dparts = _sc_deg(dstm, zeros_sl, ones_g)
    d0 = dparts[0, :N]
    d1 = dparts[1, :N]

    weights = (W1[:NUM_STATIC], W1[NUM_STATIC:NUM_STATIC + NUM_DYN],
               W1[NUM_STATIC + NUM_DYN:], b1.reshape(1, HIDDEN),
               W2, b2.reshape(1, NUM_DYN))

    pos = x[0][:, :2]
    acc0 = jnp.zeros((N, NUM_DYN), jnp.float32)

    preds = []
    y_prev8 = None
    for t in range(T):
        xt = x[t]
        static = xt[:, :NUM_STATIC]
        if y_prev8 is None:
            y08 = _pad8(jnp.clip(xt[:, NUM_STATIC:], -10.0, 10.0))
        else:
            y08 = y_prev8

        p = _sc_agg(y08, srcm, dstm, zeros_sl)
        ya, acc = _tc_stage(0.5, 1.0, static, y08, p[0, :N], p[1, :N],
                            d0, d1, y08, acc0, weights)
        p = _sc_agg(ya, srcm, dstm, zeros_sl)
        yb, acc = _tc_stage(0.5, 2.0, static, ya, p[0, :N], p[1, :N],
                            d0, d1, y08, acc, weights)
        p = _sc_agg(yb, srcm, dstm, zeros_sl)
        yc, acc = _tc_stage(1.0, 2.0, static, yb, p[0, :N], p[1, :N],
                            d0, d1, y08, acc, weights)
        p = _sc_agg(yc, srcm, dstm, zeros_sl)
        f_next = _tc_final(static, yc, p[0, :N], p[1, :N],
                           d0, d1, y08, acc, pos, weights)
        preds.append(f_next[:, :NUM_DYN])
        y_prev8 = f_next

    return jnp.stack(preds, axis=0)


# speedup vs baseline: 30.7348x; 1.2467x over previous
import functools

import jax
import jax.numpy as jnp
from jax import lax
from jax.experimental import pallas as pl
from jax.experimental.pallas import tpu as pltpu
from jax.experimental.pallas import tpu_sc as plsc

N = 100000
T = 2
NUM_STATIC = 3
NUM_DYN = 2
HIDDEN = 128
DW = 8

NC = 2
NS = 16
NW = NC * NS

G = 128
K = 8
SUPER = K * G
NITER = 98
EDGES_PER_W = SUPER * NITER
E_PAD = EDGES_PER_W * NW
ROWS_PER_W = EDGES_PER_W // G

N_PAD = 100096
SL = N_PAD // NS


def _sc_mesh():
    return plsc.VectorSubcoreMesh(core_axis_name="c", subcore_axis_name="s",
                                  num_cores=NC, num_subcores=NS)


@functools.cache
def _make_sc_agg():
    return functools.partial(
        pl.kernel,
        out_type=jax.ShapeDtypeStruct((NC, N_PAD, DW), jnp.float32),
        mesh=_sc_mesh(),
        scratch_types=[
            pltpu.VMEM((K, G), jnp.int32),
            pltpu.VMEM((K, G), jnp.int32),
            pltpu.VMEM((K, G), jnp.int32),
            pltpu.VMEM((K, G), jnp.int32),
            pltpu.VMEM((SUPER, DW), jnp.float32),
            pltpu.VMEM((SUPER, DW), jnp.float32),
            pltpu.MemorySpace.VMEM_SHARED((N_PAD, DW), jnp.float32),
            pltpu.SemaphoreType.DMA,
            pltpu.SemaphoreType.DMA,
        ],
        compiler_params=pltpu.CompilerParams(use_tc_tiling_on_sc=False),
    )(_sc_agg_body)


def _sc_agg(y8, srcm, dstm, zeros_sl):
    return _make_sc_agg()(y8, srcm, dstm, zeros_sl)


def _sc_agg_body(y_hbm, srcm, dstm, zeros_hbm, parts, src_a, dst_a, src_b,
                 dst_b, rows_a, rows_b, agg_sh, sem_a, sem_b):
    c = lax.axis_index("c")
    s = lax.axis_index("s")
    wid = s * NC + c
    base_row = wid * ROWS_PER_W

    pltpu.sync_copy(zeros_hbm, agg_sh.at[pl.ds(s * SL, SL)])
    plsc.subcore_barrier()

    def fetch(i, src_v, dst_v, rows_v, sem):
        r0 = base_row + i * K
        pltpu.sync_copy(srcm.at[pl.ds(r0, K)], src_v)
        pltpu.sync_copy(dstm.at[pl.ds(r0, K)], dst_v)
        return [
            pltpu.async_copy(y_hbm.at[src_v.at[t]],
                             rows_v.at[pl.ds(t * G, G)], sem)
            for t in range(K)
        ]

    def drain_scatter(descs, dst_v, rows_v):
        for d in descs:
            d.wait()
        for t in range(K):
            pltpu.sync_copy(rows_v.at[pl.ds(t * G, G)],
                            agg_sh.at[dst_v.at[t]], add=True)

    fetch(0, src_a, dst_a, rows_a, sem_a)

    def body(j, carry):
        i = 2 * j
        da = [pltpu.make_async_copy(y_hbm.at[src_a.at[t]],
                                    rows_a.at[pl.ds(t * G, G)], sem_a)
              for t in range(K)]
        db = fetch(i + 1, src_b, dst_b, rows_b, sem_b)
        drain_scatter(da, dst_a, rows_a)
        fetch(i + 2, src_a, dst_a, rows_a, sem_a)
        drain_scatter(db, dst_b, rows_b)
        return carry

    lax.fori_loop(0, NITER // 2 - 1, body, 0)

    da = [pltpu.make_async_copy(y_hbm.at[src_a.at[t]],
                                rows_a.at[pl.ds(t * G, G)], sem_a)
          for t in range(K)]
    db = fetch(NITER - 1, src_b, dst_b, rows_b, sem_b)
    drain_scatter(da, dst_a, rows_a)
    drain_scatter(db, dst_b, rows_b)

    plsc.subcore_barrier()
    pltpu.sync_copy(agg_sh.at[pl.ds(s * SL, SL)], parts.at[c, pl.ds(s * SL, SL)])


@functools.cache
def _make_sc_deg():
    return functools.partial(
        pl.kernel,
        out_type=jax.ShapeDtypeStruct((NC, N_PAD, DW), jnp.float32),
        mesh=_sc_mesh(),
        scratch_types=[
            pltpu.VMEM((K, G), jnp.int32),
            pltpu.VMEM((G, DW), jnp.float32),
            pltpu.MemorySpace.VMEM_SHARED((N_PAD, DW), jnp.float32),
            pltpu.SemaphoreType.DMA,
        ],
        compiler_params=pltpu.CompilerParams(use_tc_tiling_on_sc=False),
    )(_sc_deg_body)


def _sc_deg(dstm, zeros_sl, ones_g):
    return _make_sc_deg()(dstm, zeros_sl, ones_g)


def _sc_deg_body(dstm, zeros_hbm, ones_hbm, parts, dst_v, ones_v, agg_sh, sem):
    c = lax.axis_index("c")
    s = lax.axis_index("s")
    wid = s * NC + c
    base_row = wid * ROWS_PER_W

    pltpu.sync_copy(zeros_hbm, agg_sh.at[pl.ds(s * SL, SL)])
    pltpu.sync_copy(ones_hbm, ones_v)
    plsc.subcore_barrier()

    def body(i, carry):
        r0 = base_row + i * K
        pltpu.sync_copy(dstm.at[pl.ds(r0, K)], dst_v)
        for t in range(K):
            pltpu.sync_copy(ones_v, agg_sh.at[dst_v.at[t]], add=True)
        return carry

    lax.fori_loop(0, NITER, body, 0)
    plsc.subcore_barrier()
    pltpu.sync_copy(agg_sh.at[pl.ds(s * SL, SL)], parts.at[c, pl.ds(s * SL, SL)])


R = 1000


def _mlp(s, y8, p0, p1, d0, d1, w1a, w1b, w1c, b1, w2, b2):
    y = y8[:, :NUM_DYN]
    deg = jnp.maximum(d0[:, :NUM_DYN] + d1[:, :NUM_DYN], 1.0)
    a = (p0[:, :NUM_DYN] + p1[:, :NUM_DYN]) / deg
    z = (jnp.dot(s, w1a, preferred_element_type=jnp.float32)
         + jnp.dot(y, w1b, preferred_element_type=jnp.float32)
         + jnp.dot(a, w1c, preferred_element_type=jnp.float32) + b1)
    z = jnp.maximum(z, 0.0)
    return jnp.dot(z, w2, preferred_element_type=jnp.float32) + b2


def _pad8(v):
    return jnp.concatenate([v, jnp.zeros((v.shape[0], DW - NUM_DYN), v.dtype)],
                           axis=1)


def _tc_stage_body(cy, cw, s_ref, y_ref, p0_ref, p1_ref, d0_ref, d1_ref,
                   y0_ref, acc_ref, w1a_ref, w1b_ref, w1c_ref, b1_ref,
                   w2_ref, b2_ref, ynext_ref, accout_ref):
    f = _mlp(s_ref[...], y_ref[...], p0_ref[...], p1_ref[...],
             d0_ref[...], d1_ref[...], w1a_ref[...], w1b_ref[...],
             w1c_ref[...], b1_ref[...], w2_ref[...], b2_ref[...])
    ynext_ref[...] = _pad8(y0_ref[...][:, :NUM_DYN] + cy * f)
    accout_ref[...] = acc_ref[...] + cw * f


def _tc_final_body(s_ref, y_ref, p0_ref, p1_ref, d0_ref, d1_ref,
                   y0_ref, acc_ref, pos_ref, w1a_ref, w1b_ref, w1c_ref,
                   b1_ref, w2_ref, b2_ref, out_ref):
    f = _mlp(s_ref[...], y_ref[...], p0_ref[...], p1_ref[...],
             d0_ref[...], d1_ref[...], w1a_ref[...], w1b_ref[...],
             w1c_ref[...], b1_ref[...], w2_ref[...], b2_ref[...])
    pos = pos_ref[...]
    m = 0.02
    px = pos[:, 0:1]
    py = pos[:, 1:2]
    inside = (px > m) & (px < 1.0 - m) & (py > m) & (py < 1.0 - m)
    mask = jnp.where(inside, 1.0, 0.0)
    upd = mask * ((1.0 / 6.0) * (acc_ref[...] + f))
    out_ref[...] = _pad8(jnp.clip(y0_ref[...][:, :NUM_DYN] + upd, -10.0, 10.0))


def _row_spec(width):
    return pl.BlockSpec((R, width), lambda i: (i, 0))


def _full_spec(shape):
    return pl.BlockSpec(shape, lambda i: tuple(0 for _ in shape))


_W_SPECS = [
    _full_spec((NUM_STATIC, HIDDEN)),
    _full_spec((NUM_DYN, HIDDEN)),
    _full_spec((NUM_DYN, HIDDEN)),
    _full_spec((1, HIDDEN)),
    _full_spec((HIDDEN, NUM_DYN)),
    _full_spec((1, NUM_DYN)),
]

_STAGE_IN_SPECS = (
    [_row_spec(NUM_STATIC)] + [_row_spec(DW)] * 5 + [_row_spec(DW)]
    + [_row_spec(NUM_DYN)] + _W_SPECS
)


def _tc_stage(cy, cw, s, y8, p0, p1, d0, d1, y08, acc, weights):
    body = functools.partial(_tc_stage_body, cy, cw)
    out_shape = [jax.ShapeDtypeStruct((N, DW), jnp.float32),
                 jax.ShapeDtypeStruct((N, NUM_DYN), jnp.float32)]
    return pl.pallas_call(
        body,
        grid=(N // R,),
        in_specs=_STAGE_IN_SPECS,
        out_specs=[_row_spec(DW), _row_spec(NUM_DYN)],
        out_shape=out_shape,
    )(s, y8, p0, p1, d0, d1, y08, acc, *weights)


def _tc_final(s, y8, p0, p1, d0, d1, y08, acc, pos, weights):
    in_specs = (
        [_row_spec(NUM_STATIC)] + [_row_spec(DW)] * 5 + [_row_spec(DW)]
        + [_row_spec(NUM_DYN)] + [_row_spec(2)] + _W_SPECS
    )
    return pl.pallas_call(
        _tc_final_body,
        grid=(N // R,),
        in_specs=in_specs,
        out_specs=_row_spec(DW),
        out_shape=jax.ShapeDtypeStruct((N, DW), jnp.float32),
    )(s, y8, p0, p1, d0, d1, y08, acc, pos, *weights)


def kernel(x, edge_index, W1, b1, W2, b2):
    ei = edge_index.astype(jnp.int32)
    pad = E_PAD - ei.shape[1]
    src = jnp.concatenate([ei[0], jnp.zeros((pad,), jnp.int32)])
    dst = jnp.concatenate([ei[1], jnp.full((pad,), N, jnp.int32)])
    srcm = src.reshape(E_PAD // G, G)
    dstm = dst.reshape(E_PAD // G, G)

    zeros_sl = jnp.zeros((SL, DW), jnp.float32)
    ones_g = jnp.ones((G, DW), jnp.float32)

    dparts = _sc_deg(dstm, zeros_sl, ones_g)
    d0 = dparts[0, :N]
    d1 = dparts[1, :N]

    weights = (W1[:NUM_STATIC], W1[NUM_STATIC:NUM_STATIC + NUM_DYN],
               W1[NUM_STATIC + NUM_DYN:], b1.reshape(1, HIDDEN),
               W2, b2.reshape(1, NUM_DYN))

    pos = x[0][:, :2]
    acc0 = jnp.zeros((N, NUM_DYN), jnp.float32)

    preds = []
    y_prev8 = None
    for t in range(T):
        xt = x[t]
        static = xt[:, :NUM_STATIC]
        if y_prev8 is None:
            y08 = _pad8(jnp.clip(xt[:, NUM_STATIC:], -10.0, 10.0))
        else:
            y08 = y_prev8

        p = _sc_agg(y08, srcm, dstm, zeros_sl)
        ya, acc = _tc_stage(0.5, 1.0, static, y08, p[0, :N], p[1, :N],
                            d0, d1, y08, acc0, weights)
        p = _sc_agg(ya, srcm, dstm, zeros_sl)
        yb, acc = _tc_stage(0.5, 2.0, static, ya, p[0, :N], p[1, :N],
                            d0, d1, y08, acc, weights)
        p = _sc_agg(yb, srcm, dstm, zeros_sl)
        yc, acc = _tc_stage(1.0, 2.0, static, yb, p[0, :N], p[1, :N],
                            d0, d1, y08, acc, weights)
        p = _sc_agg(yc, srcm, dstm, zeros_sl)
        f_next = _tc_final(static, yc, p[0, :N], p[1, :N],
                           d0, d1, y08, acc, pos, weights)
        preds.append(f_next[:, :NUM_DYN])
        y_prev8 = f_next

    return jnp.stack(preds, axis=0)
